# baseline (speedup 1.0000x reference)
import functools

import jax
import jax.numpy as jnp
from jax import lax
from jax.experimental import pallas as pl
from jax.experimental.pallas import tpu as pltpu
from jax.experimental.pallas import tpu_sc as plsc

N = 100000
E = 6400000
D = 8
B = 4096
TGT = 3
NC, NS, L = 2, 16, 16
NW = NC * NS
WIN = 1024
NWTOT = E // WIN
NWBASE = NWTOT // NW
NWREM = NWTOT % NW
SCAT = 128
NSCAT = WIN // SCAT
NPAD = 100096
NPT = NPAD // NS
NEG = 0.01

_mesh = plsc.VectorSubcoreMesh(core_axis_name="c", subcore_axis_name="s")
_f32 = jnp.float32


@functools.partial(
    pl.kernel,
    out_type=(
        jax.ShapeDtypeStruct((NPAD,), _f32),
        jax.ShapeDtypeStruct((NPAD,), _f32),
    ),
    mesh=_mesh,
    scratch_types=[
        pltpu.VMEM_SHARED((NPAD,), _f32),
        pltpu.VMEM((WIN,), jnp.int32),
        pltpu.VMEM((WIN,), jnp.int32),
        pltpu.VMEM((WIN,), _f32),
        pltpu.VMEM((WIN,), _f32),
        pltpu.VMEM((NPT,), _f32),
        pltpu.SemaphoreType.DMA,
        pltpu.SemaphoreType.DMA,
        pltpu.SemaphoreType.DMA,
        pltpu.SemaphoreType.DMA,
    ],
    compiler_params=pltpu.CompilerParams(use_tc_tiling_on_sc=False),
)
def _deg_kernel(dst_hbm, ew_hbm, out0, out1, deg_sh, didx_a, didx_b,
                ew_a, ew_b, stage_v, semi_a, semi_b, sems_a, sems_b):
    c = lax.axis_index("c")
    s = lax.axis_index("s")
    w = s * NC + c

    def zbody(i, carry):
        stage_v[pl.ds(i * L, L)] = jnp.zeros((L,), _f32)
        return carry

    lax.fori_loop(0, NPT // L, zbody, 0)
    pltpu.sync_copy(stage_v, deg_sh.at[pl.ds(s * NPT, NPT)])
    plsc.subcore_barrier()

    cnt = NWBASE + jnp.where(w < NWREM, 1, 0)

    def in_copies(k, ew_v, didx_v, sem, make):
        base_e = k * WIN
        f = pltpu.make_async_copy if make else pltpu.async_copy
        return [
            f(ew_hbm.at[pl.ds(base_e, WIN)], ew_v, sem),
            f(dst_hbm.at[pl.ds(base_e, WIN)], didx_v, sem),
        ]

    def sc_copies(ew_v, didx_v, sem, make):
        f = pltpu.make_async_copy if make else pltpu.async_copy
        return [f(ew_v, deg_sh.at[didx_v], sem)] if make else [
            pltpu.async_copy(ew_v, deg_sh.at[didx_v], sem, add=True)]

    in_copies(w, ew_a, didx_a, semi_a, False)

    def body(i, carry):
        ia = 2 * i
        ib = ia + 1
        ka = w + ia * NW
        kb = w + ib * NW
        gb = ib < cnt
        ga2 = ia + 2 < cnt

        @pl.when(i > 0)
        def _():
            for dd in sc_copies(ew_b, didx_b, sems_b, True):
                dd.wait()

        @pl.when(gb)
        def _():
            in_copies(kb, ew_b, didx_b, semi_b, False)

        for dd in in_copies(ka, ew_a, didx_a, semi_a, True):
            dd.wait()
        sc_copies(ew_a, didx_a, sems_a, False)

        @pl.when(gb)
        def _():
            for dd in in_copies(kb, ew_b, didx_b, semi_b, True):
                dd.wait()
            sc_copies(ew_b, didx_b, sems_b, False)

        for dd in sc_copies(ew_a, didx_a, sems_a, True):
            dd.wait()

        @pl.when(ga2)
        def _():
            in_copies(ka + 2 * NW, ew_a, didx_a, semi_a, False)

        return carry

    trips = (cnt + 1) // 2
    lax.fori_loop(0, trips, body, 0)

    @pl.when(2 * trips - 1 < cnt)
    def _():
        for dd in sc_copies(ew_b, didx_b, sems_b, True):
            dd.wait()

    plsc.subcore_barrier()

    pltpu.sync_copy(deg_sh.at[pl.ds(s * NPT, NPT)], stage_v)

    @pl.when(c == 0)
    def _():
        pltpu.sync_copy(stage_v, out0.at[pl.ds(s * NPT, NPT)])

    @pl.when(c == 1)
    def _():
        pltpu.sync_copy(stage_v, out1.at[pl.ds(s * NPT, NPT)])


@functools.partial(
    pl.kernel,
    out_type=(
        jax.ShapeDtypeStruct((NPAD, D), _f32),
        jax.ShapeDtypeStruct((NPAD, D), _f32),
    ),
    mesh=_mesh,
    scratch_types=[
        pltpu.VMEM_SHARED((NPAD, D), _f32),
        pltpu.VMEM_SHARED((NPAD, D), _f32),
        pltpu.VMEM((WIN,), jnp.int32),
        pltpu.VMEM((WIN,), jnp.int32),
        pltpu.VMEM((WIN,), jnp.int32),
        pltpu.VMEM((WIN,), jnp.int32),
        pltpu.VMEM((WIN,), _f32),
        pltpu.VMEM((WIN,), _f32),
        pltpu.VMEM((WIN, D), _f32),
        pltpu.VMEM((WIN, D), _f32),
        pltpu.SemaphoreType.DMA,
        pltpu.SemaphoreType.DMA,
        pltpu.SemaphoreType.DMA,
        pltpu.SemaphoreType.DMA,
        pltpu.SemaphoreType.DMA,
        pltpu.SemaphoreType.DMA,
    ],
    compiler_params=pltpu.CompilerParams(
        needs_layout_passes=False, use_tc_tiling_on_sc=False
    ),
)
def _agg_kernel(src_hbm, dst_hbm, ew_hbm, y_hbm, yh_hbm, out0, out1,
                y_sh, t_sh, src_a, src_b, didx_a, didx_b, ew_a, ew_b,
                rows_a, rows_b, semg_a, semg_b, semi_a, semi_b, sems_a,
                sems_b):
    c = lax.axis_index("c")
    s = lax.axis_index("s")
    w = s * NC + c
    lane = lax.iota(jnp.int32, L)
    half = lane >> 3
    lane7 = lane & 7

    chunks = [(o, min(WIN, NPT - o)) for o in range(0, NPT, WIN)]
    for co, cl in chunks:
        off = s * NPT + co
        pltpu.sync_copy(y_hbm.at[pl.ds(off, cl)], rows_a.at[pl.ds(0, cl)])
        pltpu.sync_copy(rows_a.at[pl.ds(0, cl)], y_sh.at[pl.ds(off, cl)])
        pltpu.sync_copy(yh_hbm.at[pl.ds(off, cl)], rows_a.at[pl.ds(0, cl)])
        pltpu.sync_copy(rows_a.at[pl.ds(0, cl)], t_sh.at[pl.ds(off, cl)])
    plsc.subcore_barrier()

    cnt = NWBASE + jnp.where(w < NWREM, 1, 0)

    def se_copies(k, src_v, ew_v, sem, make):
        base_e = k * WIN
        f = pltpu.make_async_copy if make else pltpu.async_copy
        return [
            f(src_hbm.at[pl.ds(base_e, WIN)], src_v, sem),
            f(ew_hbm.at[pl.ds(base_e, WIN)], ew_v, sem),
        ]

    def didx_copy(k, didx_v, sem, make):
        base_e = k * WIN
        f = pltpu.make_async_copy if make else pltpu.async_copy
        return f(dst_hbm.at[pl.ds(base_e, WIN)], didx_v, sem)

    def gath(src_v, rows_v, sem, make):
        f = pltpu.make_async_copy if make else pltpu.async_copy
        return f(y_sh.at[src_v], rows_v, sem)

    def sc_copies(rows_v, didx_v, sem, make):
        f = pltpu.make_async_copy if make else pltpu.async_copy
        return [f(rows_v, t_sh.at[didx_v], sem)] if make else [
            pltpu.async_copy(rows_v, t_sh.at[didx_v], sem, add=True)]

    def scale(ew_v, rows_v):
        @plsc.parallel_loop(0, WIN // L, unroll=4)
        def gbody(j):
            jbase = j * L
            for p in range(D):
                ridx = jbase + 2 * p + half
                sc = plsc.load_gather(ew_v, [ridx])
                r = plsc.load_gather(rows_v, [ridx, lane7])
                plsc.store_scatter(rows_v, [ridx, lane7], r * sc)

    se_copies(w, src_a, ew_a, semi_a, False)
    didx_copy(w, didx_a, semi_a, False)
    se_copies(w + NW, src_b, ew_b, semi_b, False)

    def body(i, carry):
        ia = 2 * i
        ib = ia + 1
        ka = w + ia * NW
        kb = w + ib * NW
        gb = ib < cnt
        ga2 = ia + 2 < cnt
        gb2 = ib + 2 < cnt

        @pl.when(i > 0)
        def _():
            for dd in sc_copies(rows_b, didx_b, sems_b, True):
                dd.wait()

        @pl.when(gb)
        def _():
            didx_copy(kb, didx_b, semi_b, False)

        for dd in se_copies(ka, src_a, ew_a, semi_a, True):
            dd.wait()
        gath(src_a, rows_a, semg_a, False)
        didx_copy(ka, didx_a, semi_a, True).wait()
        gath(src_a, rows_a, semg_a, True).wait()

        @pl.when(gb)
        def _():
            for dd in se_copies(kb, src_b, ew_b, semi_b, True):
                dd.wait()
            didx_copy(kb, didx_b, semi_b, True).wait()
            gath(src_b, rows_b, semg_b, False)

        scale(ew_a, rows_a)
        sc_copies(rows_a, didx_a, sems_a, False)

        @pl.when(gb)
        def _():
            gath(src_b, rows_b, semg_b, True).wait()
            scale(ew_b, rows_b)

        for dd in sc_copies(rows_a, didx_a, sems_a, True):
            dd.wait()

        @pl.when(ga2)
        def _():
            se_copies(ka + 2 * NW, src_a, ew_a, semi_a, False)
            didx_copy(ka + 2 * NW, didx_a, semi_a, False)

        @pl.when(gb)
        def _():
            sc_copies(rows_b, didx_b, sems_b, False)

        @pl.when(gb2)
        def _():
            se_copies(kb + 2 * NW, src_b, ew_b, semi_b, False)

        return carry

    trips = (cnt + 1) // 2
    lax.fori_loop(0, trips, body, 0)

    @pl.when(2 * trips - 1 < cnt)
    def _():
        for dd in sc_copies(rows_b, didx_b, sems_b, True):
            dd.wait()

    plsc.subcore_barrier()

    for co, cl in chunks:
        off = s * NPT + co
        pltpu.sync_copy(t_sh.at[pl.ds(off, cl)], rows_a.at[pl.ds(0, cl)])

        @pl.when(c == 0)
        def _():
            pltpu.sync_copy(rows_a.at[pl.ds(0, cl)], out0.at[pl.ds(off, cl)])

        @pl.when(c == 1)
        def _():
            pltpu.sync_copy(rows_a.at[pl.ds(0, cl)], out1.at[pl.ds(off, cl)])


BPW = 2 * B // NW


@functools.partial(
    pl.kernel,
    out_type=jax.ShapeDtypeStruct((2 * B, D), _f32),
    mesh=_mesh,
    scratch_types=[
        pltpu.VMEM((BPW,), jnp.int32),
        pltpu.VMEM((BPW, D), _f32),
        pltpu.SemaphoreType.DMA,
    ],
    compiler_params=pltpu.CompilerParams(use_tc_tiling_on_sc=False),
)
def _gather_kernel(x_hbm, idx_hbm, out_hbm, idx_v, rows_v, sem):
    c = lax.axis_index("c")
    s = lax.axis_index("s")
    w = s * NC + c
    base = w * BPW
    pltpu.sync_copy(idx_hbm.at[pl.ds(base, BPW)], idx_v)
    pltpu.async_copy(x_hbm.at[idx_v], rows_v, sem).wait()
    pltpu.sync_copy(rows_v, out_hbm.at[pl.ds(base, BPW)])


RF = NPAD * D // 128
RD = NPAD // 128


def _prep_body(deg0_ref, deg1_ref, emb_ref, pb_ref, dinv_ref, y_ref, yh_ref):
    deg = deg0_ref[...] + deg1_ref[...] + 1.0
    dinv = lax.rsqrt(deg)
    dinv_exp = jnp.dot(dinv, pb_ref[...],
                       preferred_element_type=_f32).reshape(RF, 128)
    dinv_ref[...] = dinv_exp
    y = emb_ref[...] * dinv_exp
    y_ref[...] = y
    yh_ref[...] = y * 0.5


_prep = pl.pallas_call(
    _prep_body,
    out_shape=(
        jax.ShapeDtypeStruct((RF, 128), _f32),
        jax.ShapeDtypeStruct((RF, 128), _f32),
        jax.ShapeDtypeStruct((RF, 128), _f32),
    ),
)


def _layer_body(t0_ref, t1_ref, dinv_ref, w_ref, b_ref, xn_ref, yn_ref,
                yhn_ref):
    t = t0_ref[...] + t1_ref[...]
    agg = t * dinv_ref[...]
    h = jnp.dot(agg, w_ref[...], preferred_element_type=_f32) + b_ref[...]
    xn = jnp.where(h >= 0, h, h * NEG)
    xn_ref[...] = xn
    yn = xn * dinv_ref[...]
    yn_ref[...] = yn
    yhn_ref[...] = yn * 0.5


_layer = pl.pallas_call(
    _layer_body,
    out_shape=(
        jax.ShapeDtypeStruct((RF, 128), _f32),
        jax.ShapeDtypeStruct((RF, 128), _f32),
        jax.ShapeDtypeStruct((RF, 128), _f32),
    ),
)


def _head_body(g_ref, l1w_ref, l1b_ref, l3w_ref, l3b_ref, out_ref):
    hv = g_ref[0:B, :]
    av = g_ref[B:2 * B, :]
    z = (jnp.dot(hv, l1w_ref[0:D, :], preferred_element_type=_f32)
         + jnp.dot(av, l1w_ref[D:2 * D, :], preferred_element_type=_f32)
         + l1b_ref[...])
    z = jnp.where(z >= 0, z, z * NEG)
    z2 = jnp.dot(z, l3w_ref[...], preferred_element_type=_f32) + l3b_ref[...]
    z2 = jnp.where(z2 >= 0, z2, z2 * NEG)
    m = jnp.max(z2, axis=0, keepdims=True)
    lse = jnp.log(jnp.sum(jnp.exp(z2 - m), axis=0, keepdims=True)) + m
    out_ref[...] = z2 - lse


_head = pl.pallas_call(
    _head_body,
    out_shape=jax.ShapeDtypeStruct((B, TGT), _f32),
)

import numpy as _np

_PB = _np.zeros((128, 1024), _np.float32)
for _m in range(1024):
    _PB[(_m // 128) * 16 + (_m % 128) // 8, _m] = 1.0


def kernel(edge_index, edge_weight, home, away, emb, W1, b1, W2, b2, W3, b3,
           lin1_W, lin1_b, lin3_W, lin3_b):
    src = edge_index[0]
    dst = edge_index[1]

    deg0, deg1 = _deg_kernel(dst, edge_weight)
    emb_flat = jnp.zeros((NPAD, D), _f32).at[0:N].set(emb).reshape(RF, 128)
    dinv, y, yh = _prep(deg0.reshape(RD, 128), deg1.reshape(RD, 128),
                        emb_flat, jnp.asarray(_PB))

    eye16 = jnp.eye(16, dtype=_f32)
    for (w, b) in ((W1, b1), (W2, b2), (W3, b3)):
        t0, t1 = _agg_kernel(src, dst, edge_weight, y.reshape(NPAD, D),
                             yh.reshape(NPAD, D))
        x, y, yh = _layer(t0.reshape(RF, 128), t1.reshape(RF, 128), dinv,
                          jnp.kron(eye16, w), jnp.tile(b, 16).reshape(1, 128))

    idx = jnp.concatenate([home, away])
    g = _gather_kernel(x.reshape(NPAD, D), idx)
    out = _head(g, lin1_W, lin1_b.reshape(1, 6), lin3_W,
                lin3_b.reshape(1, TGT))
    return out


# speedup vs baseline: 1.0341x; 1.0341x over previous
import functools

import jax
import jax.numpy as jnp
from jax import lax
from jax.experimental import pallas as pl
from jax.experimental.pallas import tpu as pltpu
from jax.experimental.pallas import tpu_sc as plsc

N = 100000
E = 6400000
D = 8
B = 4096
TGT = 3
NC, NS, L = 2, 16, 16
NW = NC * NS
WIN = 1024
NWTOT = E // WIN
NWBASE = NWTOT // NW
NWREM = NWTOT % NW
SCAT = 128
NSCAT = WIN // SCAT
NPAD = 100096
NPT = NPAD // NS
NEG = 0.01

_mesh = plsc.VectorSubcoreMesh(core_axis_name="c", subcore_axis_name="s")
_f32 = jnp.float32


@functools.partial(
    pl.kernel,
    out_type=(
        jax.ShapeDtypeStruct((NPAD,), _f32),
        jax.ShapeDtypeStruct((NPAD,), _f32),
    ),
    mesh=_mesh,
    scratch_types=[
        pltpu.VMEM_SHARED((NPAD,), _f32),
        pltpu.VMEM((WIN,), jnp.int32),
        pltpu.VMEM((WIN,), jnp.int32),
        pltpu.VMEM((WIN,), _f32),
        pltpu.VMEM((WIN,), _f32),
        pltpu.VMEM((NPT,), _f32),
        pltpu.SemaphoreType.DMA,
        pltpu.SemaphoreType.DMA,
        pltpu.SemaphoreType.DMA,
        pltpu.SemaphoreType.DMA,
    ],
    compiler_params=pltpu.CompilerParams(use_tc_tiling_on_sc=False),
)
def _deg_kernel(dst_hbm, ew_hbm, out0, out1, deg_sh, didx_a, didx_b,
                ew_a, ew_b, stage_v, semi_a, semi_b, sems_a, sems_b):
    c = lax.axis_index("c")
    s = lax.axis_index("s")
    w = s * NC + c

    def zbody(i, carry):
        stage_v[pl.ds(i * L, L)] = jnp.zeros((L,), _f32)
        return carry

    lax.fori_loop(0, NPT // L, zbody, 0)
    pltpu.sync_copy(stage_v, deg_sh.at[pl.ds(s * NPT, NPT)])
    plsc.subcore_barrier()

    cnt = NWBASE + jnp.where(w < NWREM, 1, 0)

    def in_copies(k, ew_v, didx_v, sem, make):
        base_e = k * WIN
        f = pltpu.make_async_copy if make else pltpu.async_copy
        return [
            f(ew_hbm.at[pl.ds(base_e, WIN)], ew_v, sem),
            f(dst_hbm.at[pl.ds(base_e, WIN)], didx_v, sem),
        ]

    def sc_copies(ew_v, didx_v, sem, make):
        f = pltpu.make_async_copy if make else pltpu.async_copy
        return [f(ew_v, deg_sh.at[didx_v], sem)] if make else [
            pltpu.async_copy(ew_v, deg_sh.at[didx_v], sem, add=True)]

    in_copies(w, ew_a, didx_a, semi_a, False)

    def body(i, carry):
        ia = 2 * i
        ib = ia + 1
        ka = w + ia * NW
        kb = w + ib * NW
        gb = ib < cnt
        ga2 = ia + 2 < cnt

        @pl.when(i > 0)
        def _():
            for dd in sc_copies(ew_b, didx_b, sems_b, True):
                dd.wait()

        @pl.when(gb)
        def _():
            in_copies(kb, ew_b, didx_b, semi_b, False)

        for dd in in_copies(ka, ew_a, didx_a, semi_a, True):
            dd.wait()
        sc_copies(ew_a, didx_a, sems_a, False)

        @pl.when(gb)
        def _():
            for dd in in_copies(kb, ew_b, didx_b, semi_b, True):
                dd.wait()
            sc_copies(ew_b, didx_b, sems_b, False)

        for dd in sc_copies(ew_a, didx_a, sems_a, True):
            dd.wait()

        @pl.when(ga2)
        def _():
            in_copies(ka + 2 * NW, ew_a, didx_a, semi_a, False)

        return carry

    trips = (cnt + 1) // 2
    lax.fori_loop(0, trips, body, 0)

    @pl.when(2 * trips - 1 < cnt)
    def _():
        for dd in sc_copies(ew_b, didx_b, sems_b, True):
            dd.wait()

    plsc.subcore_barrier()

    pltpu.sync_copy(deg_sh.at[pl.ds(s * NPT, NPT)], stage_v)

    @pl.when(c == 0)
    def _():
        pltpu.sync_copy(stage_v, out0.at[pl.ds(s * NPT, NPT)])

    @pl.when(c == 1)
    def _():
        pltpu.sync_copy(stage_v, out1.at[pl.ds(s * NPT, NPT)])


@functools.partial(
    pl.kernel,
    out_type=(
        jax.ShapeDtypeStruct((NPAD, D), _f32),
        jax.ShapeDtypeStruct((NPAD, D), _f32),
    ),
    mesh=_mesh,
    scratch_types=[
        pltpu.VMEM_SHARED((NPAD, D), _f32),
        pltpu.VMEM_SHARED((NPAD, D), _f32),
        pltpu.VMEM((WIN,), jnp.int32),
        pltpu.VMEM((WIN,), jnp.int32),
        pltpu.VMEM((WIN,), jnp.int32),
        pltpu.VMEM((WIN,), jnp.int32),
        pltpu.VMEM((WIN,), _f32),
        pltpu.VMEM((WIN,), _f32),
        pltpu.VMEM((WIN, D), _f32),
        pltpu.VMEM((WIN, D), _f32),
        pltpu.SemaphoreType.DMA,
        pltpu.SemaphoreType.DMA,
        pltpu.SemaphoreType.DMA,
        pltpu.SemaphoreType.DMA,
        pltpu.SemaphoreType.DMA,
        pltpu.SemaphoreType.DMA,
    ],
    compiler_params=pltpu.CompilerParams(
        needs_layout_passes=False, use_tc_tiling_on_sc=False
    ),
)
def _agg_kernel(src_hbm, dst_hbm, ew_hbm, y_hbm, yh_hbm, out0, out1,
                y_sh, t_sh, src_a, src_b, didx_a, didx_b, ew_a, ew_b,
                rows_a, rows_b, semg_a, semg_b, semi_a, semi_b, sems_a,
                sems_b):
    c = lax.axis_index("c")
    s = lax.axis_index("s")
    w = s * NC + c
    lane = lax.iota(jnp.int32, L)
    half = lane >> 3
    lane7 = lane & 7

    chunks = [(o, min(WIN, NPT - o)) for o in range(0, NPT, WIN)]
    for co, cl in chunks:
        off = s * NPT + co
        pltpu.sync_copy(y_hbm.at[pl.ds(off, cl)], rows_a.at[pl.ds(0, cl)])
        pltpu.sync_copy(rows_a.at[pl.ds(0, cl)], y_sh.at[pl.ds(off, cl)])
        pltpu.sync_copy(yh_hbm.at[pl.ds(off, cl)], rows_a.at[pl.ds(0, cl)])
        pltpu.sync_copy(rows_a.at[pl.ds(0, cl)], t_sh.at[pl.ds(off, cl)])
    plsc.subcore_barrier()

    cnt = NWBASE + jnp.where(w < NWREM, 1, 0)

    def se_copies(k, src_v, ew_v, sem, make):
        base_e = k * WIN
        f = pltpu.make_async_copy if make else pltpu.async_copy
        return [
            f(src_hbm.at[pl.ds(base_e, WIN)], src_v, sem),
            f(ew_hbm.at[pl.ds(base_e, WIN)], ew_v, sem),
        ]

    def didx_copy(k, didx_v, sem, make):
        base_e = k * WIN
        f = pltpu.make_async_copy if make else pltpu.async_copy
        return f(dst_hbm.at[pl.ds(base_e, WIN)], didx_v, sem)

    def gath(src_v, rows_v, sem, make):
        f = pltpu.make_async_copy if make else pltpu.async_copy
        return f(y_sh.at[src_v], rows_v, sem)

    def sc_copies(rows_v, didx_v, sem, make):
        f = pltpu.make_async_copy if make else pltpu.async_copy
        return [f(rows_v, t_sh.at[didx_v], sem)] if make else [
            pltpu.async_copy(rows_v, t_sh.at[didx_v], sem, add=True)]

    def scale(ew_v, rows_v):
        @plsc.parallel_loop(0, WIN // L, unroll=2)
        def gbody(j):
            jbase = j * L
            for p in range(D):
                ridx = jbase + 2 * p + half
                sc = plsc.load_gather(ew_v, [ridx])
                r = plsc.load_gather(rows_v, [ridx, lane7])
                plsc.store_scatter(rows_v, [ridx, lane7], r * sc)

    se_copies(w, src_a, ew_a, semi_a, False)
    didx_copy(w, didx_a, semi_a, False)
    se_copies(w + NW, src_b, ew_b, semi_b, False)

    def body(i, carry):
        ia = 2 * i
        ib = ia + 1
        ka = w + ia * NW
        kb = w + ib * NW
        gb = ib < cnt
        ga2 = ia + 2 < cnt
        gb2 = ib + 2 < cnt

        @pl.when(i > 0)
        def _():
            for dd in sc_copies(rows_b, didx_b, sems_b, True):
                dd.wait()

        @pl.when(gb)
        def _():
            didx_copy(kb, didx_b, semi_b, False)

        for dd in se_copies(ka, src_a, ew_a, semi_a, True):
            dd.wait()
        gath(src_a, rows_a, semg_a, False)
        didx_copy(ka, didx_a, semi_a, True).wait()
        gath(src_a, rows_a, semg_a, True).wait()

        @pl.when(gb)
        def _():
            for dd in se_copies(kb, src_b, ew_b, semi_b, True):
                dd.wait()
            didx_copy(kb, didx_b, semi_b, True).wait()
            gath(src_b, rows_b, semg_b, False)

        scale(ew_a, rows_a)
        sc_copies(rows_a, didx_a, sems_a, False)

        @pl.when(gb)
        def _():
            gath(src_b, rows_b, semg_b, True).wait()
            scale(ew_b, rows_b)

        for dd in sc_copies(rows_a, didx_a, sems_a, True):
            dd.wait()

        @pl.when(ga2)
        def _():
            se_copies(ka + 2 * NW, src_a, ew_a, semi_a, False)
            didx_copy(ka + 2 * NW, didx_a, semi_a, False)

        @pl.when(gb)
        def _():
            sc_copies(rows_b, didx_b, sems_b, False)

        @pl.when(gb2)
        def _():
            se_copies(kb + 2 * NW, src_b, ew_b, semi_b, False)

        return carry

    trips = (cnt + 1) // 2
    lax.fori_loop(0, trips, body, 0)

    @pl.when(2 * trips - 1 < cnt)
    def _():
        for dd in sc_copies(rows_b, didx_b, sems_b, True):
            dd.wait()

    plsc.subcore_barrier()

    for co, cl in chunks:
        off = s * NPT + co
        pltpu.sync_copy(t_sh.at[pl.ds(off, cl)], rows_a.at[pl.ds(0, cl)])

        @pl.when(c == 0)
        def _():
            pltpu.sync_copy(rows_a.at[pl.ds(0, cl)], out0.at[pl.ds(off, cl)])

        @pl.when(c == 1)
        def _():
            pltpu.sync_copy(rows_a.at[pl.ds(0, cl)], out1.at[pl.ds(off, cl)])


BPW = 2 * B // NW


@functools.partial(
    pl.kernel,
    out_type=jax.ShapeDtypeStruct((2 * B, D), _f32),
    mesh=_mesh,
    scratch_types=[
        pltpu.VMEM((BPW,), jnp.int32),
        pltpu.VMEM((BPW, D), _f32),
        pltpu.SemaphoreType.DMA,
    ],
    compiler_params=pltpu.CompilerParams(use_tc_tiling_on_sc=False),
)
def _gather_kernel(x_hbm, idx_hbm, out_hbm, idx_v, rows_v, sem):
    c = lax.axis_index("c")
    s = lax.axis_index("s")
    w = s * NC + c
    base = w * BPW
    pltpu.sync_copy(idx_hbm.at[pl.ds(base, BPW)], idx_v)
    pltpu.async_copy(x_hbm.at[idx_v], rows_v, sem).wait()
    pltpu.sync_copy(rows_v, out_hbm.at[pl.ds(base, BPW)])


RF = NPAD * D // 128
RD = NPAD // 128


def _prep_body(deg0_ref, deg1_ref, emb_ref, pb_ref, dinv_ref, y_ref, yh_ref):
    deg = deg0_ref[...] + deg1_ref[...] + 1.0
    dinv = lax.rsqrt(deg)
    dinv_exp = jnp.dot(dinv, pb_ref[...],
                       preferred_element_type=_f32).reshape(RF, 128)
    dinv_ref[...] = dinv_exp
    y = emb_ref[...] * dinv_exp
    y_ref[...] = y
    yh_ref[...] = y * 0.5


_prep = pl.pallas_call(
    _prep_body,
    out_shape=(
        jax.ShapeDtypeStruct((RF, 128), _f32),
        jax.ShapeDtypeStruct((RF, 128), _f32),
        jax.ShapeDtypeStruct((RF, 128), _f32),
    ),
)


def _layer_body(t0_ref, t1_ref, dinv_ref, w_ref, b_ref, xn_ref, yn_ref,
                yhn_ref):
    t = t0_ref[...] + t1_ref[...]
    agg = t * dinv_ref[...]
    h = jnp.dot(agg, w_ref[...], preferred_element_type=_f32) + b_ref[...]
    xn = jnp.where(h >= 0, h, h * NEG)
    xn_ref[...] = xn
    yn = xn * dinv_ref[...]
    yn_ref[...] = yn
    yhn_ref[...] = yn * 0.5


_layer = pl.pallas_call(
    _layer_body,
    out_shape=(
        jax.ShapeDtypeStruct((RF, 128), _f32),
        jax.ShapeDtypeStruct((RF, 128), _f32),
        jax.ShapeDtypeStruct((RF, 128), _f32),
    ),
)


def _head_body(g_ref, l1w_ref, l1b_ref, l3w_ref, l3b_ref, out_ref):
    hv = g_ref[0:B, :]
    av = g_ref[B:2 * B, :]
    z = (jnp.dot(hv, l1w_ref[0:D, :], preferred_element_type=_f32)
         + jnp.dot(av, l1w_ref[D:2 * D, :], preferred_element_type=_f32)
         + l1b_ref[...])
    z = jnp.where(z >= 0, z, z * NEG)
    z2 = jnp.dot(z, l3w_ref[...], preferred_element_type=_f32) + l3b_ref[...]
    z2 = jnp.where(z2 >= 0, z2, z2 * NEG)
    m = jnp.max(z2, axis=0, keepdims=True)
    lse = jnp.log(jnp.sum(jnp.exp(z2 - m), axis=0, keepdims=True)) + m
    out_ref[...] = z2 - lse


_head = pl.pallas_call(
    _head_body,
    out_shape=jax.ShapeDtypeStruct((B, TGT), _f32),
)

import numpy as _np

_PB = _np.zeros((128, 1024), _np.float32)
for _m in range(1024):
    _PB[(_m // 128) * 16 + (_m % 128) // 8, _m] = 1.0


def kernel(edge_index, edge_weight, home, away, emb, W1, b1, W2, b2, W3, b3,
           lin1_W, lin1_b, lin3_W, lin3_b):
    src = edge_index[0]
    dst = edge_index[1]

    deg0, deg1 = _deg_kernel(dst, edge_weight)
    emb_flat = jnp.zeros((NPAD, D), _f32).at[0:N].set(emb).reshape(RF, 128)
    dinv, y, yh = _prep(deg0.reshape(RD, 128), deg1.reshape(RD, 128),
                        emb_flat, jnp.asarray(_PB))

    eye16 = jnp.eye(16, dtype=_f32)
    for (w, b) in ((W1, b1), (W2, b2), (W3, b3)):
        t0, t1 = _agg_kernel(src, dst, edge_weight, y.reshape(NPAD, D),
                             yh.reshape(NPAD, D))
        x, y, yh = _layer(t0.reshape(RF, 128), t1.reshape(RF, 128), dinv,
                          jnp.kron(eye16, w), jnp.tile(b, 16).reshape(1, 128))

    idx = jnp.concatenate([home, away])
    g = _gather_kernel(x.reshape(NPAD, D), idx)
    out = _head(g, lin1_W, lin1_b.reshape(1, 6), lin3_W,
                lin3_b.reshape(1, TGT))
    return out
